# baseline (speedup 1.0000x reference)
import functools

import jax
import jax.numpy as jnp
from jax import lax
from jax.experimental import pallas as pl
from jax.experimental.pallas import tpu as pltpu
from jax.experimental.pallas import tpu_sc as plsc

B, H, V, D = 4096, 50, 100000, 128
NC, NS = 2, 16
NW = NC * NS
LANES = 16

NBUF = 8
R_UNROLL = 10
NSLICE = 1


SEG_CHUNK = 1


def _pool_body(idx_hbm, table_hbm, out_hbm, idx_v, big_v, out_v, sem, *,
               chunks):
    wid = lax.axis_index("s") * NC + lax.axis_index("c")
    pltpu.sync_copy(idx_hbm.at[wid], idx_v)

    for g in range(NBUF):
        pltpu.async_copy(
            table_hbm.at[idx_v.at[g]], big_v.at[pl.ds(g * H, H)], sem
        )

    def chunk_body(g, _):
        base = jnp.bitwise_and(g, NBUF - 1) * H
        pltpu.make_async_copy(
            table_hbm.at[idx_v.at[g]], big_v.at[pl.ds(base, H)], sem
        ).wait()

        accs = tuple(
            big_v[base, pl.ds(c * LANES, LANES)] for c in range(D // LANES)
        )
        for r in range(1, H):
            accs = tuple(
                accs[c] + big_v[base + r, pl.ds(c * LANES, LANES)]
                for c in range(D // LANES)
            )
        for c in range(D // LANES):
            out_v[g, pl.ds(c * LANES, LANES)] = accs[c] * (1.0 / H)

        @pl.when(g + NBUF < chunks)
        def _fire():
            pltpu.async_copy(
                table_hbm.at[idx_v.at[g + NBUF]], big_v.at[pl.ds(base, H)],
                sem,
            )

        return 0

    lax.fori_loop(0, chunks, chunk_body, 0)
    pltpu.sync_copy(out_v, out_hbm.at[wid])


def _pool(click3, table, seg_per_w):
    chunks = seg_per_w // SEG_CHUNK
    mesh = plsc.VectorSubcoreMesh(core_axis_name="c", subcore_axis_name="s")
    f = pl.kernel(
        functools.partial(_pool_body, chunks=chunks),
        out_type=jax.ShapeDtypeStruct((NW, seg_per_w, D), jnp.float32),
        mesh=mesh,
        scratch_types=(
            [pltpu.VMEM((chunks, SEG_CHUNK * H), jnp.int32)]
            + [pltpu.VMEM((NBUF * H, D), jnp.float32)]
            + [pltpu.VMEM((seg_per_w, D), jnp.float32)]
            + [pltpu.SemaphoreType.DMA]
        ),
    )
    return f(click3, table)


def _mlp_body(x_ref, w1_ref, b1_ref, w2_ref, b2_ref, w3_ref, b3_ref, o_ref):
    x = x_ref[...]
    h = jnp.dot(x, w1_ref[...], preferred_element_type=jnp.float32)
    h = jnp.maximum(h + b1_ref[...], 0.0)
    h = jnp.dot(h, w2_ref[...], preferred_element_type=jnp.float32)
    h = jnp.maximum(h + b2_ref[...], 0.0)
    h = jnp.dot(h, w3_ref[...], preferred_element_type=jnp.float32)
    o_ref[...] = jnp.maximum(h + b3_ref[...], 0.0)


def _mlp(feat, W1, b1, W2, b2, W3, b3):
    rows = feat.shape[0]
    blk = min(1024, rows)
    grid = (rows // blk,)
    full = lambda shape: pl.BlockSpec(shape, lambda i: (0, 0))
    return pl.pallas_call(
        _mlp_body,
        grid=grid,
        in_specs=[
            pl.BlockSpec((blk, D), lambda i: (i, 0)),
            full(W1.shape),
            full((1, W1.shape[1])),
            full(W2.shape),
            full((1, W2.shape[1])),
            full(W3.shape),
            full((1, W3.shape[1])),
        ],
        out_specs=pl.BlockSpec((blk, W3.shape[1]), lambda i: (i, 0)),
        out_shape=jax.ShapeDtypeStruct((rows, W3.shape[1]), jnp.float32),
    )(feat, W1, b1.reshape(1, -1), W2, b2.reshape(1, -1), W3, b3.reshape(1, -1))


def kernel(click_history, table, W1, b1, W2, b2, W3, b3):
    clicks = click_history.astype(jnp.int32)
    seg_b = B // NSLICE
    seg_per_w = seg_b // NW
    outs = []
    for t in range(NSLICE):
        click3 = clicks[t * seg_b:(t + 1) * seg_b].reshape(
            NW, seg_per_w // SEG_CHUNK, SEG_CHUNK * H)
        feat = _pool(click3, table, seg_per_w).reshape(seg_b, D)
        outs.append(_mlp(feat, W1, b1, W2, b2, W3, b3))
    return outs[0] if len(outs) == 1 else jnp.concatenate(outs, axis=0)


# speedup vs baseline: 1.6931x; 1.6931x over previous
import functools

import jax
import jax.numpy as jnp
from jax import lax
from jax.experimental import pallas as pl
from jax.experimental.pallas import tpu as pltpu
from jax.experimental.pallas import tpu_sc as plsc

B, H, V, D = 4096, 50, 100000, 128
NC, NS = 2, 16
NW = NC * NS
LANES = 16

NBUF = 8
R_UNROLL = 10
NSLICE = 1


SEG_CHUNK = 1


def _pool_body(idx_hbm, table_hbm, out_hbm, idx_v, big_v, out_v, sem, *,
               chunks):
    wid = lax.axis_index("s") * NC + lax.axis_index("c")
    pltpu.sync_copy(idx_hbm.at[wid], idx_v)

    for g in range(NBUF):
        pltpu.async_copy(
            table_hbm.at[idx_v.at[g]], big_v.at[pl.ds(g * H, H)], sem
        )

    def chunk_body(g, _):
        base = jnp.bitwise_and(g, NBUF - 1) * H
        pltpu.make_async_copy(
            table_hbm.at[idx_v.at[g]], big_v.at[pl.ds(base, H)], sem
        ).wait()

        def red(rb, accs):
            for u in range(R_UNROLL):
                row = base + rb * R_UNROLL + u
                accs = tuple(
                    accs[c] + big_v[row, pl.ds(c * LANES, LANES)]
                    for c in range(D // LANES)
                )
            return accs

        accs = lax.fori_loop(
            0, H // R_UNROLL, red,
            tuple(jnp.zeros((LANES,), jnp.float32)
                  for _ in range(D // LANES)),
        )
        for c in range(D // LANES):
            out_v[g, pl.ds(c * LANES, LANES)] = accs[c] * (1.0 / H)

        @pl.when(g + NBUF < chunks)
        def _fire():
            pltpu.async_copy(
                table_hbm.at[idx_v.at[g + NBUF]], big_v.at[pl.ds(base, H)],
                sem,
            )

        return 0

    lax.fori_loop(0, chunks, chunk_body, 0)
    pltpu.sync_copy(out_v, out_hbm.at[wid])


def _pool(click3, table, seg_per_w):
    chunks = seg_per_w // SEG_CHUNK
    mesh = plsc.VectorSubcoreMesh(core_axis_name="c", subcore_axis_name="s")
    f = pl.kernel(
        functools.partial(_pool_body, chunks=chunks),
        out_type=jax.ShapeDtypeStruct((NW, seg_per_w, D), jnp.float32),
        mesh=mesh,
        scratch_types=(
            [pltpu.VMEM((chunks, SEG_CHUNK * H), jnp.int32)]
            + [pltpu.VMEM((NBUF * H, D), jnp.float32)]
            + [pltpu.VMEM((seg_per_w, D), jnp.float32)]
            + [pltpu.SemaphoreType.DMA]
        ),
    )
    return f(click3, table)


def _mlp_body(x_ref, w1_ref, b1_ref, w2_ref, b2_ref, w3_ref, b3_ref, o_ref):
    x = x_ref[...]
    h = jnp.dot(x, w1_ref[...], preferred_element_type=jnp.float32)
    h = jnp.maximum(h + b1_ref[...], 0.0)
    h = jnp.dot(h, w2_ref[...], preferred_element_type=jnp.float32)
    h = jnp.maximum(h + b2_ref[...], 0.0)
    h = jnp.dot(h, w3_ref[...], preferred_element_type=jnp.float32)
    o_ref[...] = jnp.maximum(h + b3_ref[...], 0.0)


def _mlp(feat, W1, b1, W2, b2, W3, b3):
    rows = feat.shape[0]
    blk = min(1024, rows)
    grid = (rows // blk,)
    full = lambda shape: pl.BlockSpec(shape, lambda i: (0, 0))
    return pl.pallas_call(
        _mlp_body,
        grid=grid,
        in_specs=[
            pl.BlockSpec((blk, D), lambda i: (i, 0)),
            full(W1.shape),
            full((1, W1.shape[1])),
            full(W2.shape),
            full((1, W2.shape[1])),
            full(W3.shape),
            full((1, W3.shape[1])),
        ],
        out_specs=pl.BlockSpec((blk, W3.shape[1]), lambda i: (i, 0)),
        out_shape=jax.ShapeDtypeStruct((rows, W3.shape[1]), jnp.float32),
    )(feat, W1, b1.reshape(1, -1), W2, b2.reshape(1, -1), W3, b3.reshape(1, -1))


def kernel(click_history, table, W1, b1, W2, b2, W3, b3):
    clicks = click_history.astype(jnp.int32)
    seg_b = B // NSLICE
    seg_per_w = seg_b // NW
    outs = []
    for t in range(NSLICE):
        click3 = clicks[t * seg_b:(t + 1) * seg_b].reshape(
            NW, seg_per_w // SEG_CHUNK, SEG_CHUNK * H)
        feat = _pool(click3, table, seg_per_w).reshape(seg_b, D)
        outs.append(_mlp(feat, W1, b1, W2, b2, W3, b3))
    return outs[0] if len(outs) == 1 else jnp.concatenate(outs, axis=0)
